# baseline (speedup 1.0000x reference)
import functools

import jax
import jax.numpy as jnp
from jax import lax
from jax.experimental import pallas as pl
from jax.experimental.pallas import tpu as pltpu
from jax.experimental.pallas import tpu_sc as plsc

DIM = 64
SCALE = 8.0
NC = 2
NS = 16
NW = NC * NS
CHUNK = 128
FIRES = 4
STEP = CHUNK * FIRES


@functools.lru_cache(maxsize=None)
def _build(B):
    assert B % (NW * STEP) == 0
    b_per_w = B // NW
    n_steps = b_per_w // STEP
    mesh = plsc.VectorSubcoreMesh(
        core_axis_name="c", subcore_axis_name="s", num_cores=NC, num_subcores=NS
    )

    @functools.partial(
        pl.kernel,
        out_type=jax.ShapeDtypeStruct((B, DIM), jnp.float32),
        mesh=mesh,
        scratch_types=[
            pltpu.VMEM((FIRES, CHUNK), jnp.int32),
            pltpu.VMEM((STEP, DIM), jnp.float32),
            pltpu.SemaphoreType.DMA,
        ],
        compiler_params=pltpu.CompilerParams(use_tc_tiling_on_sc=False),
    )
    def emb(idx_hbm, table_hbm, out_hbm, idx_v, rows_v, sem):
        wid = lax.axis_index("s") * NC + lax.axis_index("c")
        idx_row0 = wid * (b_per_w // CHUNK)
        base = wid * b_per_w

        def step(g, carry):
            pltpu.sync_copy(idx_hbm.at[pl.ds(idx_row0 + g * FIRES, FIRES)], idx_v)
            copies = [
                pltpu.async_copy(
                    table_hbm.at[idx_v.at[j]],
                    rows_v.at[pl.ds(j * CHUNK, CHUNK)],
                    sem,
                )
                for j in range(FIRES)
            ]
            for c in copies:
                c.wait()

            def scale_row(i, c2):
                for q in range(DIM // 16):
                    sl = pl.ds(q * 16, 16)
                    rows_v[i, sl] = rows_v[i, sl] * SCALE
                return c2

            lax.fori_loop(0, STEP, scale_row, 0)
            pltpu.sync_copy(rows_v, out_hbm.at[pl.ds(base + g * STEP, STEP)])
            return carry

        lax.fori_loop(0, n_steps, step, 0)

    return emb


def kernel(values, table):
    B = values.size
    idx2d = values.reshape(B // CHUNK, CHUNK)
    out = _build(B)(idx2d, table)
    return out.reshape(*values.shape, DIM)
